# baseline (speedup 1.0000x reference)
import jax
import jax.numpy as jnp
from jax import lax
from jax.experimental import pallas as pl
from jax.experimental.pallas import tpu as pltpu
from jax.experimental.pallas import tpu_sc as plsc

_NUM_EXPERTS = 16
_CAPACITY = 128
_CHUNK = 1024
_TPT = 256
_GRP = _TPT // 16

_DN = lax.GatherDimensionNumbers(offset_dims=(), collapsed_slice_dims=(0,),
                                 start_index_map=(0,))


def _shuf(v, idx):
    return lax.gather(v, idx[:, None], _DN, (1,),
                      mode=lax.GatherScatterMode.PROMISE_IN_BOUNDS)


def _logits_body(h_ref, wt_ref, logit_ref, logit_t_ref):
    h = h_ref[0]
    logits = jnp.dot(h, wt_ref[...], preferred_element_type=jnp.float32)
    logit_ref[0] = logits
    logit_t_ref[0] = logits.T


def _tc_logits(hidden_states, wt):
    B, S, H = hidden_states.shape
    return pl.pallas_call(
        _logits_body,
        grid=(B, S // _CHUNK),
        in_specs=[
            pl.BlockSpec((1, _CHUNK, H), lambda b, c: (b, c, 0)),
            pl.BlockSpec((H, _NUM_EXPERTS), lambda b, c: (0, 0)),
        ],
        out_specs=(
            pl.BlockSpec((1, _CHUNK, _NUM_EXPERTS), lambda b, c: (b, c, 0)),
            pl.BlockSpec((1, _NUM_EXPERTS, _CHUNK),
                         lambda b, c: (b, 0, c)),
        ),
        out_shape=(
            jax.ShapeDtypeStruct((B, S, _NUM_EXPERTS), jnp.float32),
            jax.ShapeDtypeStruct((B, _NUM_EXPERTS, S), jnp.float32),
        ),
        compiler_params=pltpu.CompilerParams(
            dimension_semantics=("arbitrary", "arbitrary")),
    )(hidden_states, wt)


_MESH = dict(core_axis_name="c", subcore_axis_name="s")


def _sc_phase1(logits_t):
    mesh = plsc.VectorSubcoreMesh(**_MESH)

    def body(lgt_hbm, pri_hbm, pm_hbm, cnt_hbm, lgt_v, pri_v, pm_v, stage_v):
        core = lax.axis_index("c")
        sub = lax.axis_index("s")
        wid = core * 16 + sub
        lanes = lax.iota(jnp.int32, 16)
        one = jnp.ones((16,), jnp.int32)
        zero = jnp.zeros((16,), jnp.int32)
        idx15 = zero + 15

        pltpu.sync_copy(lgt_hbm.at[wid], lgt_v)

        carry = [zero] * _NUM_EXPERTS
        for g in range(_GRP):
            sl = pl.ds(g * 16, 16)
            v = [lgt_v[e, sl] for e in range(_NUM_EXPERTS)]
            m = v[0]
            for e in range(1, _NUM_EXPERTS):
                m = jnp.maximum(m, v[e])
            s = jnp.exp(v[0] - m)
            for e in range(1, _NUM_EXPERTS):
                s = s + jnp.exp(v[e] - m)
            pm_v[g] = 1.0 / s
            run = zero
            oh = []
            for e in range(_NUM_EXPERTS):
                eq = jnp.where(v[e] >= m, one, zero)
                oh.append(jnp.where(run < 1, eq, zero))
                run = run + eq
            for e in range(_NUM_EXPERTS):
                p = oh[e]
                for k in (1, 2, 4, 8):
                    sh = _shuf(p, jnp.maximum(lanes - k, 0))
                    p = p + jnp.where(lanes >= k, sh, zero)
                p = p + carry[e]
                carry[e] = _shuf(p, idx15)
                pri_v[e, sl] = oh[e] * p

        tot = zero
        for e in range(_NUM_EXPERTS):
            tot = jnp.where(lanes == e, carry[e], tot)
        stage_v[0] = tot

        pltpu.sync_copy(pri_v, pri_hbm.at[wid])
        pltpu.sync_copy(pm_v, pm_hbm.at[pl.ds(wid * _GRP, _GRP)])
        pltpu.sync_copy(stage_v, cnt_hbm.at[pl.ds(wid, 1)])

    f = pl.kernel(
        body,
        mesh=mesh,
        out_type=(
            jax.ShapeDtypeStruct((32, _NUM_EXPERTS, _TPT), jnp.int32),
            jax.ShapeDtypeStruct((32 * _GRP, 16), jnp.float32),
            jax.ShapeDtypeStruct((32, _NUM_EXPERTS), jnp.int32),
        ),
        scratch_types=[
            pltpu.VMEM((_NUM_EXPERTS, _TPT), jnp.float32),
            pltpu.VMEM((_NUM_EXPERTS, _TPT), jnp.int32),
            pltpu.VMEM((_GRP, 16), jnp.float32),
            pltpu.VMEM((1, _NUM_EXPERTS), jnp.int32),
        ],
    )
    return f(logits_t)


def _sc_phase2(pri, counts):
    mesh = plsc.VectorSubcoreMesh(**_MESH)

    def body(pri_hbm, cnt_hbm, out_hbm, pri_v, out_v, cnt8_v):
        core = lax.axis_index("c")
        sub = lax.axis_index("s")
        wid = core * 16 + sub
        one = jnp.ones((16,), jnp.int32)
        zero = jnp.zeros((16,), jnp.int32)

        pltpu.sync_copy(pri_hbm.at[wid], pri_v)
        grp8 = lax.mul(lax.div(wid, 8), 8)
        pltpu.sync_copy(cnt_hbm.at[pl.ds(grp8, 8)], cnt8_v)

        kv = zero + lax.rem(wid, 8)
        base = zero
        for j in range(8):
            w = jnp.minimum(jnp.maximum(kv - j, zero), one)
            base = base + cnt8_v[j] * w
        base_splat = [_shuf(base, zero + e) for e in range(_NUM_EXPERTS)]

        for g in range(_GRP):
            sl = pl.ds(g * 16, 16)
            for e in range(_NUM_EXPERTS):
                p = pri_v[e, sl]
                out_v[e, sl] = jnp.where(
                    p > 0,
                    jnp.where(p + base_splat[e] <= _CAPACITY, one, zero),
                    zero)

        pltpu.sync_copy(out_v, out_hbm.at[wid])

    f = pl.kernel(
        body,
        mesh=mesh,
        out_type=jax.ShapeDtypeStruct((32, _NUM_EXPERTS, _TPT), jnp.int32),
        scratch_types=[
            pltpu.VMEM((_NUM_EXPERTS, _TPT), jnp.int32),
            pltpu.VMEM((_NUM_EXPERTS, _TPT), jnp.int32),
            pltpu.VMEM((8, _NUM_EXPERTS), jnp.int32),
        ],
    )
    return f(pri, counts)


def kernel(hidden_states, W):
    B, S, H = hidden_states.shape
    wt = W.T
    logits, logits_t = _tc_logits(hidden_states, wt)
    lgt = logits_t.reshape(B, _NUM_EXPERTS, S // _TPT // 2, 2, _TPT)
    lgt = lgt.transpose(0, 2, 3, 1, 4).reshape(32, _NUM_EXPERTS, _TPT)
    pri, pm, counts = _sc_phase1(lgt)
    dispatch_t = _sc_phase2(pri, counts)
    disp = dispatch_t.reshape(B, S // _TPT, _NUM_EXPERTS, _TPT)
    disp = disp.transpose(0, 1, 3, 2).reshape(B, S, _NUM_EXPERTS)
    return (disp, pm.reshape(B, S, 1), logits)


# speedup vs baseline: 1.5267x; 1.5267x over previous
import jax
import jax.numpy as jnp
from jax import lax
from jax.experimental import pallas as pl
from jax.experimental.pallas import tpu as pltpu

_NUM_EXPERTS = 16
_CAPACITY = 128.0
_CHUNK = 1024


def _cumsum_rows(x):
    k = 1
    while k < _CHUNK:
        x = x + jnp.pad(x[:-k], ((k, 0), (0, 0)))
        k *= 2
    return x


def _router_body(h_ref, wt_ref, exp_ref, pm_ref, logit_ref, carry_ref):
    c = pl.program_id(1)

    @pl.when(c == 0)
    def _():
        carry_ref[...] = jnp.zeros_like(carry_ref)

    h = h_ref[0]
    wt = wt_ref[...]
    logits = jnp.dot(h, wt, preferred_element_type=jnp.float32)
    logit_ref[0] = logits

    m = jnp.max(logits, axis=-1, keepdims=True)
    s = jnp.sum(jnp.exp(logits - m), axis=-1, keepdims=True)
    pm_ref[0] = 1.0 / s

    eq = (logits >= m).astype(jnp.bfloat16)
    rr = lax.broadcasted_iota(jnp.int32, (_NUM_EXPERTS, _NUM_EXPERTS), 0)
    cc = lax.broadcasted_iota(jnp.int32, (_NUM_EXPERTS, _NUM_EXPERTS), 1)
    upper = (rr <= cc).astype(jnp.bfloat16)
    pfx = jnp.dot(eq, upper, preferred_element_type=jnp.float32)
    ohf = (eq * (pfx <= 1.0).astype(jnp.bfloat16)).astype(jnp.float32)

    pri = _cumsum_rows(ohf) + carry_ref[...]

    keep = (pri <= _CAPACITY).astype(jnp.float32)
    exp_ref[0] = (ohf * keep).astype(jnp.int32)
    carry_ref[...] = carry_ref[...] + jnp.sum(ohf, axis=0, keepdims=True)


def kernel(hidden_states, W):
    B, S, H = hidden_states.shape
    wt = W.T
    grid = (B, S // _CHUNK)
    out_shape = (
        jax.ShapeDtypeStruct((B, S, _NUM_EXPERTS), jnp.int32),
        jax.ShapeDtypeStruct((B, S, 1), jnp.float32),
        jax.ShapeDtypeStruct((B, S, _NUM_EXPERTS), jnp.float32),
    )
    return pl.pallas_call(
        _router_body,
        grid=grid,
        in_specs=[
            pl.BlockSpec((1, _CHUNK, H), lambda b, c: (b, c, 0)),
            pl.BlockSpec((H, _NUM_EXPERTS), lambda b, c: (0, 0)),
        ],
        out_specs=(
            pl.BlockSpec((1, _CHUNK, _NUM_EXPERTS), lambda b, c: (b, c, 0)),
            pl.BlockSpec((1, _CHUNK, 1), lambda b, c: (b, c, 0)),
            pl.BlockSpec((1, _CHUNK, _NUM_EXPERTS), lambda b, c: (b, c, 0)),
        ),
        out_shape=out_shape,
        scratch_shapes=[pltpu.VMEM((1, _NUM_EXPERTS), jnp.float32)],
        compiler_params=pltpu.CompilerParams(
            dimension_semantics=("arbitrary", "arbitrary")),
    )(hidden_states, wt)
